# baseline (speedup 1.0000x reference)
import functools

import jax
import jax.numpy as jnp
from jax import lax
from jax.experimental import pallas as pl
from jax.experimental.pallas import tpu as pltpu
from jax.experimental.pallas import tpu_sc as plsc

B0 = 4096
B1 = 200
EMB = 64
TOTAL = B0 * B1
SCALE = 8.0

NC = 2
NS = 16
NW = NC * NS
PER_W = TOTAL // NW
CHUNK = 128
N_CHUNKS = PER_W // CHUNK
NSLOT = 8
GDEPTH = 4

_mesh = plsc.VectorSubcoreMesh(
    core_axis_name="c", subcore_axis_name="s", num_cores=NC, num_subcores=NS
)


@functools.partial(
    pl.kernel,
    out_type=jax.ShapeDtypeStruct((TOTAL, EMB), jnp.float32),
    mesh=_mesh,
    scratch_types=(
        [pltpu.VMEM((N_CHUNKS, CHUNK), jnp.int32)]
        + [pltpu.VMEM((NSLOT, CHUNK, EMB), jnp.float32)]
        + [pltpu.SemaphoreType.DMA] * (2 * NSLOT)
    ),
    compiler_params=pltpu.CompilerParams(use_tc_tiling_on_sc=False),
)
def _emb_lookup(idx_hbm, table_hbm, out_hbm, idx_all, rows_v, *sems):
    gsems = sems[:NSLOT]
    osems = sems[NSLOT:]
    wid = lax.axis_index("s") * NC + lax.axis_index("c")

    pltpu.sync_copy(idx_hbm.at[wid], idx_all)

    def gather_copy(i, b):
        return pltpu.make_async_copy(
            table_hbm.at[idx_all.at[i]], rows_v.at[b], gsems[b]
        )

    def out_copy(i, b):
        return pltpu.make_async_copy(
            rows_v.at[b],
            out_hbm.at[pl.ds(wid * PER_W + i * CHUNK, CHUNK)],
            osems[b],
        )

    def scale_slot(b):
        rv = rows_v.at[b]

        def sbody(r, _):
            for u in range(2):
                row = r * 2 + u
                for k in range(EMB // 16):
                    sl = pl.ds(k * 16, 16)
                    rv[row, sl] = rv[row, sl] * SCALE
            return 0

        lax.fori_loop(0, CHUNK // 2, sbody, 0, unroll=2)

    for b in range(GDEPTH):
        gather_copy(b, b).start()

    def outer(g, _):
        for bb in range(NSLOT):
            i = g * NSLOT + bb
            gather_copy(i, bb).wait()
            scale_slot(bb)
            out_copy(i, bb).start()
            j = i + GDEPTH
            b2 = (bb + GDEPTH) % NSLOT

            @pl.when(j < N_CHUNKS)
            def _():
                @pl.when(j >= NSLOT)
                def _():
                    out_copy(j - NSLOT, b2).wait()

                gather_copy(j, b2).start()

        return 0

    lax.fori_loop(0, N_CHUNKS // NSLOT, outer, 0)

    for b in range(NSLOT):
        out_copy(N_CHUNKS - NSLOT + b, b).wait()


def kernel(x, W):
    idx = x.reshape(NW, N_CHUNKS, CHUNK).astype(jnp.int32)
    out = _emb_lookup(idx, W)
    return out.reshape(B0, B1, EMB)


# speedup vs baseline: 1.0003x; 1.0003x over previous
import functools

import jax
import jax.numpy as jnp
from jax import lax
from jax.experimental import pallas as pl
from jax.experimental.pallas import tpu as pltpu
from jax.experimental.pallas import tpu_sc as plsc

B0 = 4096
B1 = 200
EMB = 64
SCALE = 8.0

TR = B1 // 8
TC = B0 // 128
NTILES = TR * TC

NC = 2
NS = 16
NW = NC * NS
TPW = NTILES // NW
NSLOT = 8
GDEPTH = 4

_mesh = plsc.VectorSubcoreMesh(
    core_axis_name="c", subcore_axis_name="s", num_cores=NC, num_subcores=NS
)


@functools.partial(
    pl.kernel,
    out_type=jax.ShapeDtypeStruct((B0, B1, EMB), jnp.float32),
    mesh=_mesh,
    scratch_types=(
        [pltpu.VMEM((TPW, 8, 128), jnp.int32)]
        + [pltpu.VMEM((NSLOT, 128, EMB), jnp.float32)]
        + [pltpu.SemaphoreType.DMA] * (2 * NSLOT)
    ),
    compiler_params=pltpu.CompilerParams(use_tc_tiling_on_sc=False),
)
def _emb_lookup(idx_hbm, table_hbm, out_hbm, idx_all, rows_v, *sems):
    gsems = sems[:NSLOT]
    osems = sems[NSLOT:]
    wid = lax.axis_index("s") * NC + lax.axis_index("c")
    t0 = wid * TPW

    pltpu.sync_copy(idx_hbm.at[pl.ds(t0, TPW)], idx_all)

    def gather_copy(g, r):
        return pltpu.make_async_copy(
            table_hbm.at[idx_all.at[g, r]], rows_v.at[r], gsems[r]
        )

    def out_copy(g, r):
        t_id = t0 + g
        tc = lax.rem(t_id, TC)
        tr = lax.div(t_id, TC)
        return pltpu.make_async_copy(
            rows_v.at[r],
            out_hbm.at[pl.ds(tc * 128, 128), tr * 8 + r],
            osems[r],
        )

    def scale_slot(r):
        rv = rows_v.at[r]

        def sbody(row2, _):
            for u in range(2):
                row = row2 * 2 + u
                for k in range(EMB // 16):
                    sl = pl.ds(k * 16, 16)
                    rv[row, sl] = rv[row, sl] * SCALE
            return 0

        lax.fori_loop(0, 128 // 2, sbody, 0, unroll=2)

    for r in range(GDEPTH):
        gather_copy(0, r).start()

    def outer(g, _):
        for r in range(NSLOT):
            gather_copy(g, r).wait()
            scale_slot(r)
            out_copy(g, r).start()
            if r < GDEPTH:
                @pl.when(g >= 1)
                def _():
                    out_copy(g - 1, r + GDEPTH).wait()

                gather_copy(g, r + GDEPTH).start()
            else:
                @pl.when(g + 1 < TPW)
                def _():
                    out_copy(g, r - GDEPTH).wait()
                    gather_copy(g + 1, r - GDEPTH).start()

        return 0

    lax.fori_loop(0, TPW, outer, 0)

    for r in range(NSLOT):
        out_copy(TPW - 1, r).wait()


def kernel(x, W):
    xv = (
        x.T.reshape(TR, 8, TC, 128)
        .transpose(0, 2, 1, 3)
        .reshape(NTILES, 8, 128)
        .astype(jnp.int32)
    )
    return _emb_lookup(xv, W)
